# baseline (speedup 1.0000x reference)
import functools

import jax
import jax.numpy as jnp
from jax import lax
from jax.experimental import pallas as pl
from jax.experimental.pallas import tpu as pltpu
from jax.experimental.pallas import tpu_sc as plsc

_B, _L, _D, _V = 4096, 50, 128, 1000
_NC, _NS, _LANES = 2, 16, 16
_NW = _NC * _NS
_ROWS_W = _B // _NW
_IDX_W = _ROWS_W * _L
_VPAD = 1024
_V_W = _VPAD // _NS


def _sc_body(x_hbm, emb_hbm, wb_hbm, out_hbm,
             x_v, emb_v, wb_v, tl_v, t_v, o_v, t_sh, sem):
    c = lax.axis_index("c")
    s = lax.axis_index("s")
    wid = s * _NC + c
    lane = lax.iota(jnp.int32, _LANES)
    zeros = lane * 0

    vstart = jnp.where(s == _NS - 1, _V - _V_W, s * _V_W)
    pltpu.sync_copy(emb_hbm.at[pl.ds(vstart * _D, _V_W * _D)], emb_v)
    pltpu.sync_copy(wb_hbm, wb_v)

    bias = plsc.load_gather(wb_v, [zeros + _D])
    rowbases = [(cch * _LANES + lane) * _D for cch in range(_V_W // _LANES)]

    def fold_step(d, accs):
        wd = plsc.load_gather(wb_v, [zeros + d])
        return tuple(
            acc + plsc.load_gather(emb_v, [rb + d]) * wd
            for acc, rb in zip(accs, rowbases)
        )

    accs = lax.fori_loop(0, _D, fold_step, (bias,) * (_V_W // _LANES))
    for cch in range(_V_W // _LANES):
        tl_v[pl.ds(cch * _LANES, _LANES)] = accs[cch] * (1.0 / _L)

    pltpu.sync_copy(tl_v, t_sh.at[pl.ds(vstart, _V_W)])
    plsc.subcore_barrier()
    pltpu.sync_copy(t_sh, t_v)

    pltpu.sync_copy(x_hbm.at[pl.ds(wid * _IDX_W, _IDX_W)], x_v)

    def row_group(g, _):
        base = (g * _LANES + lane) * _L

        def gth(l, sums):
            xi = plsc.load_gather(x_v, [base + l])
            val = plsc.load_gather(t_v, [xi])
            return sums[1:] + (sums[0] + val,)

        sums = lax.fori_loop(0, _L, gth, (jnp.zeros((_LANES,), jnp.float32),) * 2)
        acc = sums[0] + sums[1]
        o_v[pl.ds(g * _LANES, _LANES)] = 1.0 / (1.0 + jnp.exp(-acc))
        return 0

    lax.fori_loop(0, _ROWS_W // _LANES, row_group, 0)
    pltpu.sync_copy(o_v, out_hbm.at[pl.ds(wid * _ROWS_W, _ROWS_W)])


@functools.cache
def _sc_call():
    return pl.kernel(
        _sc_body,
        out_type=jax.ShapeDtypeStruct((_B,), jnp.float32),
        mesh=plsc.VectorSubcoreMesh(
            core_axis_name="c", subcore_axis_name="s",
            num_cores=_NC, num_subcores=_NS,
        ),
        scratch_types=[
            pltpu.VMEM((_IDX_W,), jnp.int32),
            pltpu.VMEM((_V_W * _D,), jnp.float32),
            pltpu.VMEM((_D + 8,), jnp.float32),
            pltpu.VMEM((_V_W,), jnp.float32),
            pltpu.VMEM((_VPAD,), jnp.float32),
            pltpu.VMEM((_ROWS_W,), jnp.float32),
            pltpu.VMEM_SHARED((_VPAD,), jnp.float32),
            pltpu.SemaphoreType.DMA,
        ],
        compiler_params=pltpu.CompilerParams(needs_layout_passes=False),
    )


def kernel(x, emb_table, W, b):
    wb = jnp.concatenate([W.reshape(_D), b, jnp.zeros((7,), jnp.float32)])
    out = _sc_call()(x.reshape(_B * _L), emb_table.reshape(_V * _D), wb)
    return out.reshape(_B, 1)


# speedup vs baseline: 1.1371x; 1.1371x over previous
import functools

import jax
import jax.numpy as jnp
from jax import lax
from jax.experimental import pallas as pl
from jax.experimental.pallas import tpu as pltpu
from jax.experimental.pallas import tpu_sc as plsc

_B, _L, _D, _V = 4096, 50, 128, 1000
_NC, _NS, _LANES = 2, 16, 16
_NW = _NC * _NS
_ROWS_W = _B // _NW
_IDX_W = _ROWS_W * _L
_VPAD = 1024
_V_W = _VPAD // _NS


def _sc_body(x_hbm, emb_hbm, wb_hbm, out_hbm,
             x_v, emb_v, wb_v, tl_v, t_v, o_v, t_sh, sem):
    c = lax.axis_index("c")
    s = lax.axis_index("s")
    wid = s * _NC + c
    lane = lax.iota(jnp.int32, _LANES)
    zeros = lane * 0

    cp_x = pltpu.async_copy(x_hbm.at[pl.ds(wid * _IDX_W, _IDX_W)], x_v, sem)

    vstart = jnp.where(s == _NS - 1, _V - _V_W, s * _V_W)
    pltpu.sync_copy(emb_hbm.at[pl.ds(vstart * _D, _V_W * _D)], emb_v)
    pltpu.sync_copy(wb_hbm, wb_v)

    bias = plsc.load_gather(wb_v, [zeros + _D])
    rowbases = [(cch * _LANES + lane) * _D for cch in range(_V_W // _LANES)]

    def fold_step(i, accs):
        for k in range(4):
            d = i * 4 + k
            wd = plsc.load_gather(wb_v, [zeros + d])
            accs = tuple(
                acc + plsc.load_gather(emb_v, [rb + d]) * wd
                for acc, rb in zip(accs, rowbases)
            )
        return accs

    accs = lax.fori_loop(0, _D // 4, fold_step, (bias,) * (_V_W // _LANES))
    for cch in range(_V_W // _LANES):
        tl_v[pl.ds(cch * _LANES, _LANES)] = accs[cch] * (1.0 / _L)

    pltpu.sync_copy(tl_v, t_sh.at[pl.ds(vstart, _V_W)])
    plsc.subcore_barrier()
    pltpu.sync_copy(t_sh, t_v)

    cp_x.wait()
    for g in range(_ROWS_W // _LANES):
        base = (g * _LANES + lane) * _L
        sums = [jnp.zeros((_LANES,), jnp.float32) for _ in range(4)]
        for l in range(_L):
            xi = plsc.load_gather(x_v, [base + l])
            sums[l % 4] = sums[l % 4] + plsc.load_gather(t_v, [xi])
        acc = (sums[0] + sums[1]) + (sums[2] + sums[3])
        o_v[pl.ds(g * _LANES, _LANES)] = 1.0 / (1.0 + jnp.exp(-acc))
    pltpu.sync_copy(o_v, out_hbm.at[pl.ds(wid * _ROWS_W, _ROWS_W)])


@functools.cache
def _sc_call():
    return pl.kernel(
        _sc_body,
        out_type=jax.ShapeDtypeStruct((_B,), jnp.float32),
        mesh=plsc.VectorSubcoreMesh(
            core_axis_name="c", subcore_axis_name="s",
            num_cores=_NC, num_subcores=_NS,
        ),
        scratch_types=[
            pltpu.VMEM((_IDX_W,), jnp.int32),
            pltpu.VMEM((_V_W * _D,), jnp.float32),
            pltpu.VMEM((_D + 8,), jnp.float32),
            pltpu.VMEM((_V_W,), jnp.float32),
            pltpu.VMEM((_VPAD,), jnp.float32),
            pltpu.VMEM((_ROWS_W,), jnp.float32),
            pltpu.VMEM_SHARED((_VPAD,), jnp.float32),
            pltpu.SemaphoreType.DMA,
        ],
        compiler_params=pltpu.CompilerParams(needs_layout_passes=False),
    )


def kernel(x, emb_table, W, b):
    wb = jnp.concatenate([W.reshape(_D), b, jnp.zeros((7,), jnp.float32)])
    out = _sc_call()(x.reshape(_B * _L), emb_table.reshape(_V * _D), wb)
    return out.reshape(_B, 1)


# speedup vs baseline: 1.1682x; 1.0273x over previous
import functools

import jax
import jax.numpy as jnp
from jax import lax
from jax.experimental import pallas as pl
from jax.experimental.pallas import tpu as pltpu
from jax.experimental.pallas import tpu_sc as plsc

_B, _L, _D, _V = 4096, 50, 128, 1000
_NC, _NS, _LANES = 2, 16, 16
_NW = _NC * _NS
_ROWS_W = _B // _NW


def _table_body(emb_ref, w_ref, b_ref, out_ref):
    s = jnp.sum(emb_ref[...] * w_ref[...], axis=1, keepdims=True)
    out_ref[...] = (s + b_ref[0, 0]) * (1.0 / _L)


def _fold_table(emb_table, W, b):
    return pl.pallas_call(
        _table_body,
        out_shape=jax.ShapeDtypeStruct((_V, 1), jnp.float32),
    )(emb_table, W, b.reshape(1, 1))


def _sc_body(x_hbm, t_hbm, out_hbm, x_v, t_v, o_v):
    wid = lax.axis_index("s") * _NC + lax.axis_index("c")
    pltpu.sync_copy(x_hbm.at[pl.ds(wid * _ROWS_W, _ROWS_W), :], x_v)
    pltpu.sync_copy(t_hbm, t_v)
    lane = lax.iota(jnp.int32, _LANES)
    zeros = lane * 0
    for g in range(_ROWS_W // _LANES):
        rows = g * _LANES + lane
        sums = [jnp.zeros((_LANES,), jnp.float32) for _ in range(4)]
        for l in range(_L):
            xi = plsc.load_gather(x_v, [rows, zeros + l])
            sums[l % 4] = sums[l % 4] + plsc.load_gather(t_v, [xi])
        acc = (sums[0] + sums[1]) + (sums[2] + sums[3])
        o_v[pl.ds(g * _LANES, _LANES)] = 1.0 / (1.0 + jnp.exp(-acc))
    pltpu.sync_copy(o_v, out_hbm.at[pl.ds(wid * _ROWS_W, _ROWS_W)])


@functools.cache
def _sc_call():
    return pl.kernel(
        _sc_body,
        out_type=jax.ShapeDtypeStruct((_B,), jnp.float32),
        mesh=plsc.VectorSubcoreMesh(
            core_axis_name="c", subcore_axis_name="s",
            num_cores=_NC, num_subcores=_NS,
        ),
        scratch_types=[
            pltpu.VMEM((_ROWS_W, _L), jnp.int32),
            pltpu.VMEM((_V,), jnp.float32),
            pltpu.VMEM((_ROWS_W,), jnp.float32),
        ],
        compiler_params=pltpu.CompilerParams(needs_layout_passes=False),
    )


def kernel(x, emb_table, W, b):
    t = _fold_table(emb_table, W, b).reshape(_V)
    out = _sc_call()(x, t)
    return out.reshape(_B, 1)


# speedup vs baseline: 1.2019x; 1.0289x over previous
import functools

import jax
import jax.numpy as jnp
from jax import lax
from jax.experimental import pallas as pl
from jax.experimental.pallas import tpu as pltpu
from jax.experimental.pallas import tpu_sc as plsc

_B, _L, _D, _V = 4096, 50, 128, 1000
_NC, _NS, _LANES = 2, 16, 16
_NW = _NC * _NS
_ROWS_W = _B // _NW


def _table_body(emb_ref, w_ref, b_ref, out_ref):
    s = jnp.sum(emb_ref[...] * w_ref[...], axis=1, keepdims=True)
    out_ref[...] = (s + b_ref[0, 0]) * (1.0 / _L)


def _fold_table(emb_table, W, b):
    return pl.pallas_call(
        _table_body,
        out_shape=jax.ShapeDtypeStruct((_V, 1), jnp.float32),
    )(emb_table, W, b.reshape(1, 1))


def _sc_body(x_hbm, t_hbm, out_hbm, x_v, t_v, o_v):
    wid = lax.axis_index("s") * _NC + lax.axis_index("c")
    pltpu.sync_copy(x_hbm.at[pl.ds(wid * _ROWS_W * _L, _ROWS_W * _L)], x_v)
    pltpu.sync_copy(t_hbm, t_v)
    lane = lax.iota(jnp.int32, _LANES)
    for g in range(_ROWS_W // _LANES):
        base = (g * _LANES + lane) * _L
        sums = [jnp.zeros((_LANES,), jnp.float32) for _ in range(4)]
        for l in range(_L):
            xi = plsc.load_gather(x_v, [base + l])
            sums[l % 4] = sums[l % 4] + plsc.load_gather(t_v, [xi])
        acc = (sums[0] + sums[1]) + (sums[2] + sums[3])
        o_v[pl.ds(g * _LANES, _LANES)] = 1.0 / (1.0 + jnp.exp(-acc))
    pltpu.sync_copy(o_v, out_hbm.at[pl.ds(wid * _ROWS_W, _ROWS_W)])


@functools.cache
def _sc_call():
    return pl.kernel(
        _sc_body,
        out_type=jax.ShapeDtypeStruct((_B,), jnp.float32),
        mesh=plsc.VectorSubcoreMesh(
            core_axis_name="c", subcore_axis_name="s",
            num_cores=_NC, num_subcores=_NS,
        ),
        scratch_types=[
            pltpu.VMEM((_ROWS_W * _L,), jnp.int32),
            pltpu.VMEM((_V,), jnp.float32),
            pltpu.VMEM((_ROWS_W,), jnp.float32),
        ],
        compiler_params=pltpu.CompilerParams(needs_layout_passes=False),
    )


def kernel(x, emb_table, W, b):
    t = _fold_table(emb_table, W, b).reshape(_V)
    out = _sc_call()(x.reshape(_B * _L), t)
    return out.reshape(_B, 1)
